# Initial kernel scaffold; baseline (speedup 1.0000x reference)
import jax
import jax.numpy as jnp
from jax.experimental import pallas as pl


def kernel(inputs, pos_table):
    raise NotImplementedError("write your pallas kernel here")


# speedup vs baseline: 1.6741x; 1.6741x over previous
import jax
import jax.numpy as jnp
from jax.experimental import pallas as pl


_BLK_S = 512


def _add_kernel(x_ref, p_ref, o_ref):
    o_ref[...] = x_ref[...] + p_ref[...]


def kernel(inputs, pos_table):
    batch, seq, dim = inputs.shape
    grid = (seq // _BLK_S, batch)
    return pl.pallas_call(
        _add_kernel,
        grid=grid,
        in_specs=[
            pl.BlockSpec((1, _BLK_S, dim), lambda i, b: (b, i, 0)),
            pl.BlockSpec((_BLK_S, dim), lambda i, b: (i, 0)),
        ],
        out_specs=pl.BlockSpec((1, _BLK_S, dim), lambda i, b: (b, i, 0)),
        out_shape=jax.ShapeDtypeStruct((batch, seq, dim), inputs.dtype),
    )(inputs, pos_table)


# speedup vs baseline: 1.8482x; 1.1040x over previous
import jax
import jax.numpy as jnp
from jax.experimental import pallas as pl


_BLK_S = 1024


def _add_kernel(x_ref, p_ref, o_ref):
    o_ref[...] = x_ref[...] + p_ref[...]


def kernel(inputs, pos_table):
    batch, seq, dim = inputs.shape
    grid = (seq // _BLK_S, batch)
    return pl.pallas_call(
        _add_kernel,
        grid=grid,
        in_specs=[
            pl.BlockSpec((1, _BLK_S, dim), lambda i, b: (b, i, 0)),
            pl.BlockSpec((_BLK_S, dim), lambda i, b: (i, 0)),
        ],
        out_specs=pl.BlockSpec((1, _BLK_S, dim), lambda i, b: (b, i, 0)),
        out_shape=jax.ShapeDtypeStruct((batch, seq, dim), inputs.dtype),
    )(inputs, pos_table)


# speedup vs baseline: 1.9688x; 1.0652x over previous
import jax
import jax.numpy as jnp
from jax.experimental import pallas as pl


_BLK_S = 2048


def _add_kernel(x_ref, p_ref, o_ref):
    o_ref[...] = x_ref[...] + p_ref[...]


def kernel(inputs, pos_table):
    batch, seq, dim = inputs.shape
    grid = (seq // _BLK_S, batch)
    return pl.pallas_call(
        _add_kernel,
        grid=grid,
        in_specs=[
            pl.BlockSpec((1, _BLK_S, dim), lambda i, b: (b, i, 0)),
            pl.BlockSpec((_BLK_S, dim), lambda i, b: (i, 0)),
        ],
        out_specs=pl.BlockSpec((1, _BLK_S, dim), lambda i, b: (b, i, 0)),
        out_shape=jax.ShapeDtypeStruct((batch, seq, dim), inputs.dtype),
    )(inputs, pos_table)
